# initial kernel scaffold (unmeasured)
import jax
import jax.numpy as jnp
from jax import lax
from jax.experimental import pallas as pl
from jax.experimental.pallas import tpu as pltpu

N_DEV = 4
B = 8
NB = 512
BS = 16
H = 8
D = 128
P_LOCAL = 512
T_LOCAL = P_LOCAL * BS

NEG = -1e30


def kernel(Q, K, V, bt, lens):
    def body(q_ref, k_ref, v_ref, bt_ref, lens_ref, out_ref,
             k_buf, v_buf, local_ref, comm_ref, copy_sems, send_sems, recv_sems):
        my_pos = lax.axis_index("i")
        lo = my_pos * P_LOCAL

        barrier_sem = pltpu.get_barrier_semaphore()
        for o in range(1, N_DEV):
            pl.semaphore_signal(
                barrier_sem, inc=1,
                device_id=((my_pos + o) % N_DEV,),
                device_id_type=pl.DeviceIdType.MESH,
            )
        pl.semaphore_wait(barrier_sem, N_DEV - 1)

        pages = lax.broadcasted_iota(jnp.int32, (P_LOCAL, NB), 0) + lo
        slots = lax.broadcasted_iota(jnp.int32, (P_LOCAL, NB), 1)
        ctok_rows = []
        for i in range(B):
            bt_row = bt_ref[pl.ds(i, 1), :]
            valid = slots < lens_ref[i]
            eq = (pages == bt_row) & valid
            cnt = jnp.sum(eq.astype(jnp.float32), axis=1, keepdims=True)
            ctok_rows.append(
                jnp.reshape(jnp.broadcast_to(cnt, (P_LOCAL, BS)), (1, T_LOCAL))
            )
        ctok = jnp.concatenate(ctok_rows, axis=0)
        selected = ctok > 0.0

        q_all = q_ref[:, 0, :, :]
        scale = D ** -0.5

        for h in range(H):
            ck = pltpu.make_async_copy(
                k_ref.at[:, :, pl.ds(h, 1), :], k_buf, copy_sems.at[0])
            cv = pltpu.make_async_copy(
                v_ref.at[:, :, pl.ds(h, 1), :], v_buf, copy_sems.at[1])
            ck.start()
            cv.start()
            ck.wait()
            cv.wait()

            k_h = jnp.reshape(k_buf[...], (T_LOCAL, D)).astype(jnp.bfloat16)
            v_h = jnp.reshape(v_buf[...], (T_LOCAL, D)).astype(jnp.bfloat16)
            q_h = q_all[:, h, :].astype(jnp.bfloat16)

            s = lax.dot_general(
                q_h, k_h, (((1,), (1,)), ((), ())),
                preferred_element_type=jnp.float32,
            ) * scale
            s = jnp.where(selected, s, NEG)
            m_h = jnp.max(s, axis=1, keepdims=True)
            w = ctok * jnp.exp(s - m_h)
            l_h = jnp.sum(w, axis=1, keepdims=True)
            o_h = lax.dot_general(
                w.astype(jnp.bfloat16), v_h, (((1,), (0,)), ((), ())),
                preferred_element_type=jnp.float32,
            )

            local_ref[pl.ds(h * B, B), :] = o_h
            local_ref[pl.ds(B * H + h * B, B), :] = jnp.broadcast_to(m_h, (B, D))
            local_ref[pl.ds(2 * B * H + h * B, B), :] = jnp.broadcast_to(l_h, (B, D))

        rdmas = []
        for o in range(1, N_DEV):
            rdma = pltpu.make_async_remote_copy(
                src_ref=local_ref,
                dst_ref=comm_ref.at[o - 1],
                send_sem=send_sems.at[o - 1],
                recv_sem=recv_sems.at[o - 1],
                device_id=((my_pos + o) % N_DEV,),
                device_id_type=pl.DeviceIdType.MESH,
            )
            rdma.start()
            rdmas.append(rdma)
        for r in rdmas:
            r.wait_send()
        for r in rdmas:
            r.wait_recv()

        parts = [local_ref[...]] + [comm_ref[o] for o in range(N_DEV - 1)]
        for h in range(H):
            ms = [p[B * H + h * B:B * H + (h + 1) * B, :] for p in parts]
            m_glob = ms[0]
            for m in ms[1:]:
                m_glob = jnp.maximum(m_glob, m)
            o_acc = jnp.zeros((B, D), jnp.float32)
            l_acc = jnp.zeros((B, D), jnp.float32)
            for p, m in zip(parts, ms):
                alpha = jnp.exp(m - m_glob)
                o_acc = o_acc + alpha * p[h * B:(h + 1) * B, :]
                l_acc = l_acc + alpha * p[2 * B * H + h * B:2 * B * H + (h + 1) * B, :]
            out_ref[:, 0, h, :] = o_acc / l_acc

    return pl.pallas_call(
        body,
        out_shape=jax.ShapeDtypeStruct((B, 1, H, D), jnp.float32),
        in_specs=[
            pl.BlockSpec(memory_space=pltpu.VMEM),
            pl.BlockSpec(memory_space=pltpu.ANY),
            pl.BlockSpec(memory_space=pltpu.ANY),
            pl.BlockSpec(memory_space=pltpu.VMEM),
            pl.BlockSpec(memory_space=pltpu.SMEM),
        ],
        out_specs=pl.BlockSpec(memory_space=pltpu.VMEM),
        scratch_shapes=[
            pltpu.VMEM((P_LOCAL, BS, 1, D), jnp.float32),
            pltpu.VMEM((P_LOCAL, BS, 1, D), jnp.float32),
            pltpu.VMEM((3 * B * H, D), jnp.float32),
            pltpu.VMEM((N_DEV - 1, 3 * B * H, D), jnp.float32),
            pltpu.SemaphoreType.DMA((2,)),
            pltpu.SemaphoreType.DMA((N_DEV - 1,)),
            pltpu.SemaphoreType.DMA((N_DEV - 1,)),
        ],
        compiler_params=pltpu.CompilerParams(collective_id=0),
    )(Q, K, V, bt, lens)


# baseline (device time: 109192 ns/iter reference)
import jax
import jax.numpy as jnp
from jax import lax
from jax.experimental import pallas as pl
from jax.experimental.pallas import tpu as pltpu

N_DEV = 4
B = 8
NB = 512
BS = 16
H = 8
D = 128
P_LOCAL = 512
T_LOCAL = P_LOCAL * BS


def kernel(Q, K, V, bt, lens):
    my_pos = lax.axis_index("i")
    btl = bt - my_pos * P_LOCAL
    valid = jnp.arange(NB)[None, :] < lens[:, None]
    onehot = (btl[:, :, None] == jnp.arange(P_LOCAL)[None, None, :]) & valid[:, :, None]
    cnt = jnp.sum(onehot.astype(jnp.float32), axis=1)
    ctok = jnp.repeat(cnt, BS, axis=1)
    qt = jnp.transpose(Q[:, 0], (1, 0, 2))

    def body(qt_ref, k_ref, v_ref, ctok_ref, out_ref,
             k_buf, v_buf, local_ref, comm_ref, copy_sems, send_sems, recv_sems):
        h = pl.program_id(0)
        me = lax.axis_index("i")

        @pl.when(h == 0)
        def _barrier():
            barrier_sem = pltpu.get_barrier_semaphore()
            for o in range(1, N_DEV):
                pl.semaphore_signal(
                    barrier_sem, inc=1,
                    device_id=((me + o) % N_DEV,),
                    device_id_type=pl.DeviceIdType.MESH,
                )
            pl.semaphore_wait(barrier_sem, N_DEV - 1)

        ck = pltpu.make_async_copy(
            k_ref.at[:, :, pl.ds(h, 1), :], k_buf, copy_sems.at[0])
        cv = pltpu.make_async_copy(
            v_ref.at[:, :, pl.ds(h, 1), :], v_buf, copy_sems.at[1])
        ck.start()
        cv.start()
        ck.wait()
        cv.wait()

        k_h = jnp.reshape(k_buf[...], (T_LOCAL, D)).astype(jnp.bfloat16)
        v_h = jnp.reshape(v_buf[...], (T_LOCAL, D)).astype(jnp.bfloat16)
        q_h = jnp.reshape(qt_ref[pl.ds(h, 1)], (B, D)).astype(jnp.bfloat16)
        ctok = ctok_ref[...]

        s = lax.dot_general(
            q_h, k_h, (((1,), (1,)), ((), ())),
            preferred_element_type=jnp.float32,
        ) * (D ** -0.5)
        m_h = jnp.max(s, axis=1, keepdims=True)
        w = ctok * jnp.exp(s - m_h)
        l_h = jnp.sum(w, axis=1, keepdims=True)
        o_h = lax.dot_general(
            w.astype(jnp.bfloat16), v_h, (((1,), (0,)), ((), ())),
            preferred_element_type=jnp.float32,
        )

        local_ref[pl.ds(h * B, B), :] = o_h
        local_ref[pl.ds(B * H + h * B, B), :] = jnp.broadcast_to(m_h, (B, D))
        local_ref[pl.ds(2 * B * H + h * B, B), :] = jnp.broadcast_to(l_h, (B, D))

        @pl.when(h == H - 1)
        def _exchange_and_combine():
            rdmas = []
            for o in range(1, N_DEV):
                rdma = pltpu.make_async_remote_copy(
                    src_ref=local_ref,
                    dst_ref=comm_ref.at[o - 1],
                    send_sem=send_sems.at[o - 1],
                    recv_sem=recv_sems.at[o - 1],
                    device_id=((me + o) % N_DEV,),
                    device_id_type=pl.DeviceIdType.MESH,
                )
                rdma.start()
                rdmas.append(rdma)
            for r in rdmas:
                r.wait_send()
            for r in rdmas:
                r.wait_recv()

            parts = [local_ref[...]] + [comm_ref[o] for o in range(N_DEV - 1)]
            for hh in range(H):
                ms = [p[B * H + hh * B:B * H + (hh + 1) * B, :] for p in parts]
                m_glob = ms[0]
                for m in ms[1:]:
                    m_glob = jnp.maximum(m_glob, m)
                o_acc = jnp.zeros((B, D), jnp.float32)
                l_acc = jnp.zeros((B, D), jnp.float32)
                for p, m in zip(parts, ms):
                    alpha = jnp.exp(m - m_glob)
                    o_acc = o_acc + alpha * p[hh * B:(hh + 1) * B, :]
                    l_acc = l_acc + alpha * p[2 * B * H + hh * B:2 * B * H + (hh + 1) * B, :]
                out_ref[:, 0, hh, :] = o_acc / l_acc

    return pl.pallas_call(
        body,
        grid=(H,),
        out_shape=jax.ShapeDtypeStruct((B, 1, H, D), jnp.float32),
        in_specs=[
            pl.BlockSpec(memory_space=pltpu.VMEM),
            pl.BlockSpec(memory_space=pl.ANY),
            pl.BlockSpec(memory_space=pl.ANY),
            pl.BlockSpec(memory_space=pltpu.VMEM),
        ],
        out_specs=pl.BlockSpec(memory_space=pltpu.VMEM),
        scratch_shapes=[
            pltpu.VMEM((P_LOCAL, BS, 1, D), jnp.float32),
            pltpu.VMEM((P_LOCAL, BS, 1, D), jnp.float32),
            pltpu.VMEM((3 * B * H, D), jnp.float32),
            pltpu.VMEM((N_DEV - 1, 3 * B * H, D), jnp.float32),
            pltpu.SemaphoreType.DMA((2,)),
            pltpu.SemaphoreType.DMA((N_DEV - 1,)),
            pltpu.SemaphoreType.DMA((N_DEV - 1,)),
        ],
        compiler_params=pltpu.CompilerParams(
            dimension_semantics=("arbitrary",),
            collective_id=0,
        ),
    )(qt, K, V, ctok)


# device time: 85525 ns/iter; 1.2767x vs baseline; 1.2767x over previous
import jax
import jax.numpy as jnp
from jax import lax
from jax.experimental import pallas as pl
from jax.experimental.pallas import tpu as pltpu

N_DEV = 4
B = 8
NB = 512
BS = 16
H = 8
D = 128
P_LOCAL = 512
T_LOCAL = P_LOCAL * BS


def kernel(Q, K, V, bt, lens):
    my_pos = lax.axis_index("i")
    btl = bt - my_pos * P_LOCAL
    valid = jnp.arange(NB)[None, :] < lens[:, None]
    onehot = (btl[:, :, None] == jnp.arange(P_LOCAL)[None, None, :]) & valid[:, :, None]
    cnt = jnp.sum(onehot.astype(jnp.float32), axis=1)
    ctok = jnp.repeat(cnt, BS, axis=1)
    qt = jnp.transpose(Q[:, 0], (1, 0, 2))

    def body(qt_ref, k_ref, v_ref, ctok_ref, out_ref,
             k_buf, v_buf, local_ref, comm_ref, copy_sems, send_sems, recv_sems):
        h = pl.program_id(0)
        me = lax.axis_index("i")

        @pl.when(h == 0)
        def _barrier():
            barrier_sem = pltpu.get_barrier_semaphore()
            for o in range(1, N_DEV):
                pl.semaphore_signal(
                    barrier_sem, inc=1,
                    device_id=((me + o) % N_DEV,),
                    device_id_type=pl.DeviceIdType.MESH,
                )
            pl.semaphore_wait(barrier_sem, N_DEV - 1)

        slot = h % 2
        nslot = (h + 1) % 2

        def _dma_pair(head, s):
            ck = pltpu.make_async_copy(
                k_ref.at[:, :, pl.ds(head, 1), :], k_buf.at[s], copy_sems.at[0, s])
            cv = pltpu.make_async_copy(
                v_ref.at[:, :, pl.ds(head, 1), :], v_buf.at[s], copy_sems.at[1, s])
            return ck, cv

        @pl.when(h == 0)
        def _prime():
            ck, cv = _dma_pair(h, slot)
            ck.start()
            cv.start()

        @pl.when(h < H - 1)
        def _prefetch_next():
            ck, cv = _dma_pair(h + 1, nslot)
            ck.start()
            cv.start()

        ck, cv = _dma_pair(h, slot)
        ck.wait()
        cv.wait()

        k_h = jnp.reshape(k_buf[slot], (T_LOCAL, D)).astype(jnp.bfloat16)
        v_h = jnp.reshape(v_buf[slot], (T_LOCAL, D)).astype(jnp.bfloat16)
        q_h = jnp.reshape(qt_ref[pl.ds(h, 1)], (B, D)).astype(jnp.bfloat16)
        ctok = ctok_ref[...]

        s = lax.dot_general(
            q_h, k_h, (((1,), (1,)), ((), ())),
            preferred_element_type=jnp.float32,
        ) * (D ** -0.5)
        m_h = jnp.max(s, axis=1, keepdims=True)
        w = ctok * jnp.exp(s - m_h)
        l_h = jnp.sum(w, axis=1, keepdims=True)
        o_h = lax.dot_general(
            w.astype(jnp.bfloat16), v_h, (((1,), (0,)), ((), ())),
            preferred_element_type=jnp.float32,
        )

        local_ref[pl.ds(h * B, B), :] = o_h
        local_ref[pl.ds(B * H + h * B, B), :] = jnp.broadcast_to(m_h, (B, D))
        local_ref[pl.ds(2 * B * H + h * B, B), :] = jnp.broadcast_to(l_h, (B, D))

        @pl.when(h == H - 1)
        def _exchange_and_combine():
            rdmas = []
            for o in range(1, N_DEV):
                rdma = pltpu.make_async_remote_copy(
                    src_ref=local_ref,
                    dst_ref=comm_ref.at[o - 1],
                    send_sem=send_sems.at[o - 1],
                    recv_sem=recv_sems.at[o - 1],
                    device_id=((me + o) % N_DEV,),
                    device_id_type=pl.DeviceIdType.MESH,
                )
                rdma.start()
                rdmas.append(rdma)
            for r in rdmas:
                r.wait_send()
            for r in rdmas:
                r.wait_recv()

            parts = [local_ref[...]] + [comm_ref[o] for o in range(N_DEV - 1)]
            for hh in range(H):
                ms = [p[B * H + hh * B:B * H + (hh + 1) * B, :] for p in parts]
                m_glob = ms[0]
                for m in ms[1:]:
                    m_glob = jnp.maximum(m_glob, m)
                o_acc = jnp.zeros((B, D), jnp.float32)
                l_acc = jnp.zeros((B, D), jnp.float32)
                for p, m in zip(parts, ms):
                    alpha = jnp.exp(m - m_glob)
                    o_acc = o_acc + alpha * p[hh * B:(hh + 1) * B, :]
                    l_acc = l_acc + alpha * p[2 * B * H + hh * B:2 * B * H + (hh + 1) * B, :]
                out_ref[:, 0, hh, :] = o_acc / l_acc

    return pl.pallas_call(
        body,
        grid=(H,),
        out_shape=jax.ShapeDtypeStruct((B, 1, H, D), jnp.float32),
        in_specs=[
            pl.BlockSpec(memory_space=pltpu.VMEM),
            pl.BlockSpec(memory_space=pl.ANY),
            pl.BlockSpec(memory_space=pl.ANY),
            pl.BlockSpec(memory_space=pltpu.VMEM),
        ],
        out_specs=pl.BlockSpec(memory_space=pltpu.VMEM),
        scratch_shapes=[
            pltpu.VMEM((2, P_LOCAL, BS, 1, D), jnp.float32),
            pltpu.VMEM((2, P_LOCAL, BS, 1, D), jnp.float32),
            pltpu.VMEM((3 * B * H, D), jnp.float32),
            pltpu.VMEM((N_DEV - 1, 3 * B * H, D), jnp.float32),
            pltpu.SemaphoreType.DMA((2, 2)),
            pltpu.SemaphoreType.DMA((N_DEV - 1,)),
            pltpu.SemaphoreType.DMA((N_DEV - 1,)),
        ],
        compiler_params=pltpu.CompilerParams(
            dimension_semantics=("arbitrary",),
            collective_id=0,
        ),
    )(qt, K, V, ctok)
